# baseline (speedup 1.0000x reference)
import functools

import jax
import jax.numpy as jnp
from jax import lax
from jax.experimental import pallas as pl
from jax.experimental.pallas import tpu as pltpu
from jax.experimental.pallas import tpu_sc as plsc

B = 1024
L = 200
D = 128
NC = 2
NS = 16
NW = NC * NS
ROWS_PER_W = (B * L) // NW
SUB = 128
NSUB = ROWS_PER_W // SUB


def _make_kernel():
  mesh = plsc.VectorSubcoreMesh(
      core_axis_name="c", subcore_axis_name="s", num_cores=NC, num_subcores=NS
  )

  @functools.partial(
      pl.kernel,
      out_type=jax.ShapeDtypeStruct((NW, NSUB, SUB, D), jnp.float32),
      mesh=mesh,
      scratch_types=[
          pltpu.VMEM((NSUB, SUB), jnp.int32),
          pltpu.VMEM((NSUB, SUB), jnp.int32),
          pltpu.VMEM((NSUB, SUB), jnp.int32),
          pltpu.VMEM((SUB, D), jnp.float32),
          pltpu.SemaphoreType.DMA,
      ],
  )
  def bert_embed(ids_hbm, sids_hbm, pids_hbm, tok_hbm, pos_hbm, seg_hbm,
                 out_hbm, idx_v, sidx_v, lidx_v, buf, sem):
    wid = lax.axis_index("s") * NC + lax.axis_index("c")
    pltpu.sync_copy(ids_hbm.at[wid], idx_v)
    pltpu.sync_copy(sids_hbm.at[wid], sidx_v)
    pltpu.sync_copy(pids_hbm.at[wid], lidx_v)

    def step(j, carry):
      pltpu.async_copy(pos_hbm.at[lidx_v.at[j]], buf, sem).wait()
      pltpu.async_copy(seg_hbm.at[sidx_v.at[j]], buf, sem, add=True).wait()
      pltpu.async_copy(tok_hbm.at[idx_v.at[j]], buf, sem, add=True).wait()
      pltpu.sync_copy(buf, out_hbm.at[wid, j])
      return carry

    lax.fori_loop(0, NSUB, step, 0)

  return bert_embed


_bert_embed = _make_kernel()


@jax.jit
def kernel(input_ids, segment_ids, token_table, position_table, segment_table):
  ids = input_ids.astype(jnp.int32).reshape(NW, NSUB, SUB)
  sids = segment_ids.astype(jnp.int32).reshape(NW, NSUB, SUB)
  pids = jnp.broadcast_to(
      jnp.arange(L, dtype=jnp.int32), (B, L)
  ).reshape(NW, NSUB, SUB)
  out = _bert_embed(ids, sids, pids, token_table, position_table,
                    segment_table)
  return out.reshape(B, L, D)
